# Initial kernel scaffold; baseline (speedup 1.0000x reference)
import jax
import jax.numpy as jnp
from jax.experimental import pallas as pl


def kernel(d_noised, threshold):
    raise NotImplementedError("write your pallas kernel here")


# speedup vs baseline: 3.9673x; 3.9673x over previous
import jax
import jax.numpy as jnp
from jax.experimental import pallas as pl
from jax.experimental.pallas import tpu as pltpu

M = 4096
N = 4096
BI = 256


def _stencil_kernel(thr_ref, top_ref, bot_ref, d_ref, out_ref):
    i = pl.program_id(0)
    t = thr_ref[0]
    c = d_ref[...]
    top = top_ref[0]
    bot = bot_ref[0]

    up = jnp.concatenate([top, c[:-1, :]], axis=0)
    down = jnp.concatenate([c[1:, :], bot], axis=0)

    def shift_l(v):
        return jnp.concatenate([v[:, -1:], v[:, :-1]], axis=1)

    def shift_r(v):
        return jnp.concatenate([v[:, 1:], v[:, :1]], axis=1)

    cA = jnp.abs(shift_l(up) - c) <= t
    cB = jnp.abs(shift_l(down) - c) <= t
    cC = jnp.abs(shift_r(down) - c) <= t
    cD = jnp.abs(shift_r(up) - c) <= t

    gi = i * BI + jax.lax.broadcasted_iota(jnp.int32, (BI, N), 0)
    jj = jax.lax.broadcasted_iota(jnp.int32, (BI, N), 1)
    i_ge1 = gi >= 1
    i_lt = gi <= M - 2
    j_ge1 = jj >= 1
    j_lt = jj <= N - 2

    mA = i_ge1 & j_ge1
    mB = i_ge1 & i_lt & j_ge1 & j_lt
    mC = i_lt & j_lt

    combined = (mA & cA) | (mB & cB) | (mC & cC) | cD
    out_ref[...] = combined.astype(jnp.int32).T > 0


def kernel(d_noised, threshold):
    nb = M // BI
    starts = jnp.arange(nb) * BI
    top_rows = jnp.take(d_noised, (starts - 1) % M, axis=0).reshape(nb, 1, N)
    bot_rows = jnp.take(d_noised, (starts + BI) % M, axis=0).reshape(nb, 1, N)
    thr = jnp.reshape(threshold, (1,))

    out = pl.pallas_call(
        _stencil_kernel,
        grid=(nb,),
        in_specs=[
            pl.BlockSpec(memory_space=pltpu.SMEM),
            pl.BlockSpec((1, 1, N), lambda i: (i, 0, 0)),
            pl.BlockSpec((1, 1, N), lambda i: (i, 0, 0)),
            pl.BlockSpec((BI, N), lambda i: (i, 0)),
        ],
        out_specs=pl.BlockSpec((N, BI), lambda i: (0, i)),
        out_shape=jax.ShapeDtypeStruct((N, M), jnp.bool_),
        compiler_params=pltpu.CompilerParams(
            dimension_semantics=("arbitrary",),
        ),
    )(thr, top_rows, bot_rows, d_noised)
    return out


# speedup vs baseline: 4.5492x; 1.1467x over previous
import jax
import jax.numpy as jnp
from jax.experimental import pallas as pl
from jax.experimental.pallas import tpu as pltpu

M = 4096
N = 4096
BI = 256


def _stencil_kernel(thr_ref, topA_ref, topD_ref, bot_ref, d_ref, out_ref):
    i = pl.program_id(0)
    t = thr_ref[0]
    c = d_ref[...]
    topA = topA_ref[0]
    topD = topD_ref[0]
    bot = bot_ref[0]

    inf = jnp.float32(jnp.inf)
    infcol = jnp.full((BI, 1), inf, jnp.float32)
    infcol1 = jnp.full((1, 1), inf, jnp.float32)

    cL = jnp.concatenate([infcol, c[:, :-1]], axis=1)
    tAL = jnp.concatenate([infcol1, topA[:, :-1]], axis=1)
    bL = jnp.concatenate([infcol1, bot[:, :-1]], axis=1)
    cR = jnp.concatenate([c[:, 1:], c[:, :1]], axis=1)
    tDR = jnp.concatenate([topD[:, 1:], topD[:, :1]], axis=1)
    cRC = jnp.concatenate([c[:, 1:], infcol], axis=1)
    bRC = jnp.concatenate([bot[:, 1:], infcol1], axis=1)

    upAL = jnp.concatenate([tAL, cL[:-1, :]], axis=0)
    dnL = jnp.concatenate([cL[1:, :], bL], axis=0)
    dnRC = jnp.concatenate([cRC[1:, :], bRC], axis=0)
    upDR = jnp.concatenate([tDR, cR[:-1, :]], axis=0)

    cA = jnp.abs(upAL - c) <= t
    cB = jnp.abs(dnL - c) <= t
    cC = jnp.abs(dnRC - c) <= t
    cD = jnp.abs(upDR - c) <= t

    row = jax.lax.broadcasted_iota(jnp.int32, (BI, N), 0)
    lanes = jax.lax.broadcasted_iota(jnp.int32, (BI, N), 1)
    mB = (row >= 1 - i * BI) & (lanes <= N - 2)

    combined = (cA | (cB & mB)) | (cC | cD)
    out_ref[...] = combined.astype(jnp.int8).T != 0


def kernel(d_noised, threshold):
    nb = M // BI
    starts = jnp.arange(nb) * BI
    inf_row = jnp.full((1, N), jnp.inf, jnp.float32)
    topD_rows = jnp.take(d_noised, (starts - 1) % M, axis=0)
    topA_rows = jnp.concatenate([inf_row, topD_rows[1:]], axis=0)
    bot_rows = jnp.concatenate(
        [jnp.take(d_noised, starts[:-1] + BI, axis=0), inf_row], axis=0
    )
    thr = jnp.reshape(threshold, (1,))

    out = pl.pallas_call(
        _stencil_kernel,
        grid=(nb,),
        in_specs=[
            pl.BlockSpec(memory_space=pltpu.SMEM),
            pl.BlockSpec((1, 1, N), lambda i: (i, 0, 0)),
            pl.BlockSpec((1, 1, N), lambda i: (i, 0, 0)),
            pl.BlockSpec((1, 1, N), lambda i: (i, 0, 0)),
            pl.BlockSpec((BI, N), lambda i: (i, 0)),
        ],
        out_specs=pl.BlockSpec((N, BI), lambda i: (0, i)),
        out_shape=jax.ShapeDtypeStruct((N, M), jnp.bool_),
        compiler_params=pltpu.CompilerParams(
            dimension_semantics=("arbitrary",),
        ),
    )(
        thr,
        topA_rows.reshape(nb, 1, N),
        topD_rows.reshape(nb, 1, N),
        bot_rows.reshape(nb, 1, N),
        d_noised,
    )
    return out
